# Initial kernel scaffold; baseline (speedup 1.0000x reference)
import jax
import jax.numpy as jnp
from jax.experimental import pallas as pl


def kernel(u, grid, inc):
    raise NotImplementedError("write your pallas kernel here")


# speedup vs baseline: 246.5652x; 246.5652x over previous
import jax
import jax.numpy as jnp
from jax import lax
from jax.experimental import pallas as pl
from jax.experimental.pallas import tpu as pltpu
from jax.experimental.pallas import tpu_sc as plsc

BATCH = 524288
DIM = 32
NINC = 1000

NC = 2
NS = 16
L = 16
NW = NC * NS
SPW = BATCH // NW
BLK = 256
NBLK = SPW // BLK


def _sc_body(u_hbm, grid_hbm, inc_hbm, x_hbm, det_hbm,
             grid_v, inc_v, u_v, x_v, det_v):
    wid = lax.axis_index("s") * NC + lax.axis_index("c")
    base = wid * SPW

    pltpu.sync_copy(grid_hbm, grid_v)
    pltpu.sync_copy(inc_hbm, inc_v)

    lane32 = lax.iota(jnp.int32, L) * DIM
    ninc_f = jnp.full((L,), float(NINC), jnp.float32)
    ninc_m1 = jnp.full((L,), NINC - 1, jnp.int32)

    @pl.loop(0, NBLK)
    def _blk(b):
        start = base + b * BLK
        pltpu.sync_copy(u_hbm.at[pl.ds(start * DIM, BLK * DIM)], u_v)

        @pl.loop(0, BLK // L)
        def _grp(g):
            eoff = lane32 + g * (L * DIM)
            prod = jnp.ones((L,), jnp.float32)
            for d in range(DIM):
                uv = plsc.load_gather(u_v, [eoff + d])
                un = uv * ninc_f
                iu = un.astype(jnp.int32)
                du = un - iu.astype(jnp.float32)
                iu_c = jnp.minimum(iu, ninc_m1)
                gv = plsc.load_gather(grid_v, [iu_c + d * (NINC + 1)])
                hv = plsc.load_gather(inc_v, [iu_c + d * NINC])
                xv = gv + hv * du
                plsc.store_scatter(x_v, [eoff + d], xv)
                prod = prod * (hv * ninc_f)
            det_v[pl.ds(g * L, L)] = prod

        pltpu.sync_copy(x_v, x_hbm.at[pl.ds(start * DIM, BLK * DIM)])
        pltpu.sync_copy(det_v, det_hbm.at[pl.ds(start, BLK)])


_sc_call = pl.kernel(
    _sc_body,
    out_type=[
        jax.ShapeDtypeStruct((BATCH * DIM,), jnp.float32),
        jax.ShapeDtypeStruct((BATCH,), jnp.float32),
    ],
    mesh=plsc.VectorSubcoreMesh(core_axis_name="c", subcore_axis_name="s"),
    compiler_params=pltpu.CompilerParams(needs_layout_passes=False),
    scratch_types=[
        pltpu.VMEM((DIM * (NINC + 1),), jnp.float32),
        pltpu.VMEM((DIM * NINC,), jnp.float32),
        pltpu.VMEM((BLK * DIM,), jnp.float32),
        pltpu.VMEM((BLK * DIM,), jnp.float32),
        pltpu.VMEM((BLK,), jnp.float32),
    ],
)


def _log_body(d_ref, o_ref):
    o_ref[...] = jnp.log(d_ref[...])


_log_call = pl.pallas_call(
    _log_body,
    out_shape=jax.ShapeDtypeStruct((BATCH,), jnp.float32),
)


def kernel(u, grid, inc):
    x_flat, det = _sc_call(u.reshape(-1), grid.reshape(-1), inc.reshape(-1))
    return x_flat.reshape(BATCH, DIM), _log_call(det)


# speedup vs baseline: 312.6358x; 1.2680x over previous
import jax
import jax.numpy as jnp
from jax import lax
from jax.experimental import pallas as pl
from jax.experimental.pallas import tpu as pltpu
from jax.experimental.pallas import tpu_sc as plsc

BATCH = 524288
DIM = 32
NINC = 1000

NC = 2
NS = 16
L = 16
NW = NC * NS
SPW = BATCH // NW
BLK = 512
NBLK = SPW // BLK


def _sc_body(u_hbm, grid_hbm, inc_hbm, x_hbm, det_hbm,
             grid_v, inc_v, u_v, x_v, det_v):
    wid = lax.axis_index("s") * NC + lax.axis_index("c")
    base = wid * SPW

    pltpu.sync_copy(grid_hbm, grid_v)
    pltpu.sync_copy(inc_hbm, inc_v)

    lane32 = lax.iota(jnp.int32, L) * DIM
    ninc_f = jnp.full((L,), float(NINC), jnp.float32)
    ninc_m1 = jnp.full((L,), NINC - 1, jnp.int32)

    @pl.loop(0, NBLK)
    def _blk(b):
        start = base + b * BLK
        pltpu.sync_copy(u_hbm.at[pl.ds(start * DIM, BLK * DIM)], u_v)

        @plsc.parallel_loop(0, BLK // L, unroll=2)
        def _grp(g):
            eoff = lane32 + g * (L * DIM)
            prod = jnp.ones((L,), jnp.float32)
            for d in range(DIM):
                uv = plsc.load_gather(u_v, [eoff + d])
                un = uv * ninc_f
                iu = un.astype(jnp.int32)
                du = un - iu.astype(jnp.float32)
                iu_c = jnp.minimum(iu, ninc_m1)
                gv = plsc.load_gather(grid_v, [iu_c + d * (NINC + 1)])
                hv = plsc.load_gather(inc_v, [iu_c + d * NINC])
                xv = gv + hv * du
                plsc.store_scatter(x_v, [eoff + d], xv)
                prod = prod * (hv * ninc_f)
            det_v[pl.ds(g * L, L)] = prod

        pltpu.sync_copy(x_v, x_hbm.at[pl.ds(start * DIM, BLK * DIM)])
        pltpu.sync_copy(det_v, det_hbm.at[pl.ds(start, BLK)])


_sc_call = pl.kernel(
    _sc_body,
    out_type=[
        jax.ShapeDtypeStruct((BATCH * DIM,), jnp.float32),
        jax.ShapeDtypeStruct((BATCH,), jnp.float32),
    ],
    mesh=plsc.VectorSubcoreMesh(core_axis_name="c", subcore_axis_name="s"),
    compiler_params=pltpu.CompilerParams(needs_layout_passes=False),
    scratch_types=[
        pltpu.VMEM((DIM * (NINC + 1),), jnp.float32),
        pltpu.VMEM((DIM * NINC,), jnp.float32),
        pltpu.VMEM((BLK * DIM,), jnp.float32),
        pltpu.VMEM((BLK * DIM,), jnp.float32),
        pltpu.VMEM((BLK,), jnp.float32),
    ],
)


def _log_body(d_ref, o_ref):
    o_ref[...] = jnp.log(d_ref[...])


_log_call = pl.pallas_call(
    _log_body,
    out_shape=jax.ShapeDtypeStruct((BATCH,), jnp.float32),
)


def kernel(u, grid, inc):
    x_flat, det = _sc_call(u.reshape(-1), grid.reshape(-1), inc.reshape(-1))
    return x_flat.reshape(BATCH, DIM), _log_call(det)


# speedup vs baseline: 395.1100x; 1.2638x over previous
import jax
import jax.numpy as jnp
from jax import lax
from jax.experimental import pallas as pl
from jax.experimental.pallas import tpu as pltpu
from jax.experimental.pallas import tpu_sc as plsc

BATCH = 524288
DIM = 32
NINC = 1000

NC = 2
NS = 16
L = 16
NW = NC * NS
SPW = BATCH // NW
BLK = 512
NBLK = SPW // BLK


def _sc_body(u_hbm, grid_hbm, inc_hbm, x_hbm, det_hbm,
             grid_v, inc_v, u_v, x_v, det_v):
    wid = lax.axis_index("s") * NC + lax.axis_index("c")
    base = wid * SPW

    pltpu.sync_copy(grid_hbm, grid_v)
    pltpu.sync_copy(inc_hbm, inc_v)

    lane32 = lax.iota(jnp.int32, L) * DIM
    ninc_f = jnp.full((L,), float(NINC), jnp.float32)
    ninc_m1 = jnp.full((L,), NINC - 1, jnp.int32)

    @pl.loop(0, NBLK)
    def _blk(b):
        start = base + b * BLK
        pltpu.sync_copy(u_hbm.at[pl.ds(start * DIM, BLK * DIM)], u_v)

        @plsc.parallel_loop(0, BLK // L, unroll=2)
        def _grp(g):
            eoff = lane32 + g * (L * DIM)

            @plsc.parallel_loop(0, DIM, unroll=8,
                                carry=jnp.ones((L,), jnp.float32))
            def _dim(d, prod):
                uv = plsc.load_gather(u_v, [eoff + d])
                un = uv * ninc_f
                iu = un.astype(jnp.int32)
                du = un - iu.astype(jnp.float32)
                iu_c = jnp.minimum(iu, ninc_m1)
                gv = plsc.load_gather(grid_v, [iu_c + d * (NINC + 1)])
                hv = plsc.load_gather(inc_v, [iu_c + d * NINC])
                xv = gv + hv * du
                plsc.store_scatter(x_v, [eoff + d], xv)
                return prod * (hv * ninc_f)

            det_v[pl.ds(g * L, L)] = _dim

        pltpu.sync_copy(x_v, x_hbm.at[pl.ds(start * DIM, BLK * DIM)])
        pltpu.sync_copy(det_v, det_hbm.at[pl.ds(start, BLK)])


_sc_call = pl.kernel(
    _sc_body,
    out_type=[
        jax.ShapeDtypeStruct((BATCH * DIM,), jnp.float32),
        jax.ShapeDtypeStruct((BATCH,), jnp.float32),
    ],
    mesh=plsc.VectorSubcoreMesh(core_axis_name="c", subcore_axis_name="s"),
    compiler_params=pltpu.CompilerParams(needs_layout_passes=False),
    scratch_types=[
        pltpu.VMEM((DIM * (NINC + 1),), jnp.float32),
        pltpu.VMEM((DIM * NINC,), jnp.float32),
        pltpu.VMEM((BLK * DIM,), jnp.float32),
        pltpu.VMEM((BLK * DIM,), jnp.float32),
        pltpu.VMEM((BLK,), jnp.float32),
    ],
)


def _log_body(d_ref, o_ref):
    o_ref[...] = jnp.log(d_ref[...])


_log_call = pl.pallas_call(
    _log_body,
    out_shape=jax.ShapeDtypeStruct((BATCH,), jnp.float32),
)


def kernel(u, grid, inc):
    x_flat, det = _sc_call(u.reshape(-1), grid.reshape(-1), inc.reshape(-1))
    return x_flat.reshape(BATCH, DIM), _log_call(det)
